# baseline (speedup 1.0000x reference)
import jax
import jax.numpy as jnp
from jax import lax
from jax.experimental import pallas as pl
from jax.experimental.pallas import tpu as pltpu
from jax.experimental.pallas import tpu_sc as plsc

VOCAB = 1000000
EMBED_DIM = 32
BATCH = 4096
HIST = 50

_NC = 2
_NS = 16
_NW = _NC * _NS

_CH = 128
_NCH = HIST
_NBUF = 5
_NBUF_R = 6

_VT = 7813
_VT_FULL = _VT - 1
_SR = 250016


def _relayout_body(tt_hbm, tail_hbm, scratch_hbm, in_v, out_v, i_sems, o_sems):
    wid = lax.axis_index("s") * _NC + lax.axis_index("c")

    iota = lax.broadcasted_iota(jnp.int32, (16,), 0)
    lo16 = iota
    hi16 = iota + 16

    def start_in(vt, b):
        pltpu.async_copy(
            tt_hbm.at[:, pl.ds(vt * 128, 128)], in_v.at[b], i_sems.at[b]
        )

    def wait_in(vt, b):
        pltpu.make_async_copy(
            tt_hbm.at[:, pl.ds(vt * 128, 128)], in_v.at[b], i_sems.at[b]
        ).wait()

    def transpose(b, ncols):
        for qq in range(ncols // 16):
            qs = [qq * 4 + i for i in range(4)]
            vecs = [
                plsc.load_gather(
                    in_v.at[b],
                    [lo16 if k % 2 == 0 else hi16,
                     jnp.full((16,), q * 4 + k // 2, jnp.int32)],
                )
                for q in qs
                for k in range(8)
            ]
            for i, q in enumerate(qs):
                for k in range(8):
                    out_v[b, q, pl.ds(k * 16, 16)] = vecs[i * 8 + k]

    def start_out(vt, b, nrows):
        pltpu.async_copy(
            out_v.at[b, pl.ds(0, nrows)],
            scratch_hbm.at[pl.ds(vt * 32, nrows)],
            o_sems.at[b],
        )

    def wait_out(vt, b, nrows):
        pltpu.make_async_copy(
            out_v.at[b, pl.ds(0, nrows)],
            scratch_hbm.at[pl.ds(vt * 32, nrows)],
            o_sems.at[b],
        ).wait()

    n_my = (_VT_FULL - wid + _NW - 1) // _NW

    def vt_of(k):
        return k * _NW + wid

    for b in range(_NBUF_R):
        @pl.when(b < n_my)
        def _():
            start_in(vt_of(b), b)

    def step(g, _):
        for b in range(_NBUF_R):
            k = g * _NBUF_R + b

            @pl.when(k < n_my)
            def _():
                vt = vt_of(k)
                wait_in(vt, b)

                @pl.when(k >= _NBUF_R)
                def _():
                    wait_out(vt_of(k - _NBUF_R), b, 32)

                transpose(b, 128)

                @pl.when(k + _NBUF_R < n_my)
                def _():
                    start_in(vt_of(k + _NBUF_R), b)

                start_out(vt, b, 32)
        return _

    _N_K = (_VT_FULL + _NW - 1) // _NW
    lax.fori_loop(0, (_N_K + _NBUF_R - 1) // _NBUF_R, step, None)

    for b in range(_NBUF_R):
        kl = n_my - 1 - ((n_my - 1 - b) % _NBUF_R)

        @pl.when(kl >= 0)
        def _():
            wait_out(vt_of(kl), b, 32)

    @pl.when(wid == 0)
    def _():
        pltpu.sync_copy(tail_hbm, in_v.at[0, pl.ds(0, 16)])
        pltpu.sync_copy(
            in_v.at[0, pl.ds(0, 16)],
            scratch_hbm.at[pl.ds(_VT_FULL * 32, 16)],
        )


def _gather_body(
    x_hbm, table_hbm, out_hbm, idx_v, idx4_v, rows_v, t_v, g_sems, o_sems
):
    wid = lax.axis_index("s") * _NC + lax.axis_index("c")

    pltpu.sync_copy(x_hbm.at[:, wid], idx_v)

    iotas = [
        lax.broadcasted_iota(jnp.int32, (16,), 0) + g * 16 for g in range(8)
    ]
    cols = [jnp.full((16,), d, jnp.int32) for d in range(EMBED_DIM)]

    def shift_chunk(j, _):
        for g in range(8):
            v = idx_v[j, pl.ds(g * 16, 16)]
            idx4_v[j, pl.ds(g * 16, 16)] = v >> 2
        return _

    lax.fori_loop(0, _NCH, shift_chunk, None)

    def start_gather(j, b):
        pltpu.async_copy(table_hbm.at[idx4_v.at[j]], rows_v.at[b], g_sems.at[b])

    def wait_gather(j, b):
        pltpu.make_async_copy(
            table_hbm.at[idx4_v.at[j]], rows_v.at[b], g_sems.at[b]
        ).wait()

    def transpose(j, b):
        for g in range(8):
            sub32 = (idx_v[j, pl.ds(g * 16, 16)] & 3) << 5
            vecs = [
                plsc.load_gather(rows_v.at[b], [iotas[g], sub32 | cols[d]])
                for d in range(EMBED_DIM)
            ]
            for d in range(EMBED_DIM):
                t_v[b, d // 8, d % 8, pl.ds(g * 16, 16)] = vecs[d]

    def start_out(j, b):
        pltpu.async_copy(t_v.at[b], out_hbm.at[j, :, wid], o_sems.at[b])

    def wait_out(j, b):
        pltpu.make_async_copy(
            t_v.at[b], out_hbm.at[j, :, wid], o_sems.at[b]
        ).wait()

    for b in range(_NBUF):
        start_gather(b, b)

    def step(g, _):
        for b in range(_NBUF):
            j = g * _NBUF + b
            wait_gather(j, b)

            @pl.when(j >= _NBUF)
            def _():
                wait_out(j - _NBUF, b)

            transpose(j, b)

            @pl.when(j + _NBUF < _NCH)
            def _():
                start_gather(j + _NBUF, b)

            start_out(j, b)
        return _

    lax.fori_loop(0, _NCH // _NBUF, step, None)

    for b in range(_NBUF):
        wait_out(_NCH - _NBUF + b, b)


def _sc_relayout(tt, tail4):
    kern = pl.kernel(
        _relayout_body,
        out_type=jax.ShapeDtypeStruct((_SR, 128), jnp.float32),
        mesh=plsc.VectorSubcoreMesh(core_axis_name="c", subcore_axis_name="s"),
        scratch_types=[
            pltpu.VMEM((_NBUF_R, 32, 128), jnp.float32),
            pltpu.VMEM((_NBUF_R, 32, 128), jnp.float32),
            pltpu.SemaphoreType.DMA((_NBUF_R,)),
            pltpu.SemaphoreType.DMA((_NBUF_R,)),
        ],
        compiler_params=pltpu.CompilerParams(
            use_tc_tiling_on_sc=True, needs_layout_passes=False
        ),
    )
    return kern(tt, tail4)


def _sc_gather(x3, table4):
    kern = pl.kernel(
        _gather_body,
        out_type=jax.ShapeDtypeStruct((HIST, 4, _NW, 8, _CH), jnp.float32),
        mesh=plsc.VectorSubcoreMesh(core_axis_name="c", subcore_axis_name="s"),
        scratch_types=[
            pltpu.VMEM((_NCH, _CH), jnp.int32),
            pltpu.VMEM((_NCH, _CH), jnp.int32),
            pltpu.VMEM((_NBUF, _CH, 128), jnp.float32),
            pltpu.VMEM((_NBUF, 4, 8, _CH), jnp.float32),
            pltpu.SemaphoreType.DMA((_NBUF,)),
            pltpu.SemaphoreType.DMA((_NBUF,)),
        ],
        compiler_params=pltpu.CompilerParams(
            use_tc_tiling_on_sc=False, needs_layout_passes=False
        ),
    )
    return kern(x3, table4)


def kernel(x, embedding_matrix):
    x3 = x.astype(jnp.int32).T.reshape(HIST, _NW, _CH)
    tail4 = embedding_matrix[_VT_FULL * 128:].reshape(16, 128)
    table4 = _sc_relayout(embedding_matrix.T, tail4)
    out5 = _sc_gather(x3, table4)
    return out5.transpose(2, 4, 0, 1, 3).reshape(BATCH, HIST, EMBED_DIM)


# speedup vs baseline: 1.0216x; 1.0216x over previous
import jax
import jax.numpy as jnp
from jax import lax
from jax.experimental import pallas as pl
from jax.experimental.pallas import tpu as pltpu
from jax.experimental.pallas import tpu_sc as plsc

VOCAB = 1000000
EMBED_DIM = 32
BATCH = 4096
HIST = 50

_NC = 2
_NS = 16
_NW = _NC * _NS

_CH = 128
_NCH = HIST
_NBUF = 2
_NBUF_R = 3

_VT = 7813
_VT_FULL = _VT - 1
_SR = 250016


def _relayout_body(tt_hbm, tail_hbm, scratch_hbm, in_v, out_v, i_sems, o_sems):
    wid = lax.axis_index("s") * _NC + lax.axis_index("c")

    iota = lax.broadcasted_iota(jnp.int32, (16,), 0)
    lo16 = iota
    hi16 = iota + 16

    def start_in(vt, b):
        pltpu.async_copy(
            tt_hbm.at[:, pl.ds(vt * 128, 128)],
            in_v.at[b, :, pl.ds(0, 128)],
            i_sems.at[b],
        )

    def wait_in(vt, b):
        pltpu.make_async_copy(
            tt_hbm.at[:, pl.ds(vt * 128, 128)],
            in_v.at[b, :, pl.ds(0, 128)],
            i_sems.at[b],
        ).wait()

    def transpose(b, ncols):
        for qq in range(ncols // 16):
            qs = [qq * 4 + i for i in range(4)]
            vecs = [
                plsc.load_gather(
                    in_v.at[b],
                    [lo16 if k % 2 == 0 else hi16,
                     jnp.full((16,), q * 4 + k // 2, jnp.int32)],
                )
                for q in qs
                for k in range(8)
            ]
            for i, q in enumerate(qs):
                for k in range(8):
                    out_v[b, q, pl.ds(k * 16, 16)] = vecs[i * 8 + k]

    def start_out(vt, b, nrows):
        pltpu.async_copy(
            out_v.at[b, pl.ds(0, nrows)],
            scratch_hbm.at[pl.ds(vt * 32, nrows)],
            o_sems.at[b],
        )

    def wait_out(vt, b, nrows):
        pltpu.make_async_copy(
            out_v.at[b, pl.ds(0, nrows)],
            scratch_hbm.at[pl.ds(vt * 32, nrows)],
            o_sems.at[b],
        ).wait()

    n_my = (_VT_FULL - wid + _NW - 1) // _NW

    def vt_of(k):
        return k * _NW + wid

    for b in range(_NBUF_R):
        @pl.when(b < n_my)
        def _():
            start_in(vt_of(b), b)

    def step(g, _):
        for b in range(_NBUF_R):
            k = g * _NBUF_R + b

            @pl.when(k < n_my)
            def _():
                vt = vt_of(k)
                wait_in(vt, b)

                @pl.when(k >= _NBUF_R)
                def _():
                    wait_out(vt_of(k - _NBUF_R), b, 32)

                transpose(b, 128)

                @pl.when(k + _NBUF_R < n_my)
                def _():
                    start_in(vt_of(k + _NBUF_R), b)

                start_out(vt, b, 32)
        return _

    _N_K = (_VT_FULL + _NW - 1) // _NW
    lax.fori_loop(0, (_N_K + _NBUF_R - 1) // _NBUF_R, step, None)

    for b in range(_NBUF_R):
        kl = n_my - 1 - ((n_my - 1 - b) % _NBUF_R)

        @pl.when(kl >= 0)
        def _():
            wait_out(vt_of(kl), b, 32)

    @pl.when(wid == 0)
    def _():
        pltpu.sync_copy(tail_hbm, in_v.at[0, pl.ds(0, 16), pl.ds(0, 128)])
        pltpu.sync_copy(
            in_v.at[0, pl.ds(0, 16), pl.ds(0, 128)],
            scratch_hbm.at[pl.ds(_VT_FULL * 32, 16)],
        )


def _gather_body(
    x_hbm, table_hbm, out_hbm, idx_v, idx4_v, rows_v, t_v, g_sems, o_sems
):
    wid = lax.axis_index("s") * _NC + lax.axis_index("c")

    pltpu.sync_copy(x_hbm.at[:, wid], idx_v)

    iotas = [
        lax.broadcasted_iota(jnp.int32, (16,), 0) + g * 16 for g in range(8)
    ]
    cols = [jnp.full((16,), d, jnp.int32) for d in range(EMBED_DIM)]

    def shift_chunk(j, _):
        for g in range(8):
            v = idx_v[j, pl.ds(g * 16, 16)]
            idx4_v[j, pl.ds(g * 16, 16)] = v >> 2
        return _

    lax.fori_loop(0, _NCH, shift_chunk, None)

    def start_gather(j, b):
        pltpu.async_copy(table_hbm.at[idx4_v.at[j]], rows_v.at[b], g_sems.at[b])

    def wait_gather(j, b):
        pltpu.make_async_copy(
            table_hbm.at[idx4_v.at[j]], rows_v.at[b], g_sems.at[b]
        ).wait()

    def transpose(j, b):
        for g in range(8):
            sub32 = (idx_v[j, pl.ds(g * 16, 16)] & 3) << 5
            vecs = [
                plsc.load_gather(rows_v.at[b], [iotas[g], sub32 | cols[d]])
                for d in range(EMBED_DIM)
            ]
            for d in range(EMBED_DIM):
                t_v[b, d // 8, d % 8, pl.ds(g * 16, 16)] = vecs[d]

    def start_out(j, b):
        pltpu.async_copy(t_v.at[b], out_hbm.at[j, :, wid], o_sems.at[b])

    def wait_out(j, b):
        pltpu.make_async_copy(
            t_v.at[b], out_hbm.at[j, :, wid], o_sems.at[b]
        ).wait()

    for b in range(_NBUF):
        start_gather(b, b)

    def step(g, _):
        for b in range(_NBUF):
            j = g * _NBUF + b
            wait_gather(j, b)

            @pl.when(j >= _NBUF)
            def _():
                wait_out(j - _NBUF, b)

            transpose(j, b)

            @pl.when(j + _NBUF < _NCH)
            def _():
                start_gather(j + _NBUF, b)

            start_out(j, b)
        return _

    lax.fori_loop(0, _NCH // _NBUF, step, None)

    for b in range(_NBUF):
        wait_out(_NCH - _NBUF + b, b)


def _sc_relayout(tt, tail4):
    kern = pl.kernel(
        _relayout_body,
        out_type=jax.ShapeDtypeStruct((_SR, 128), jnp.float32),
        mesh=plsc.VectorSubcoreMesh(core_axis_name="c", subcore_axis_name="s"),
        scratch_types=[
            pltpu.VMEM((_NBUF_R, 32, 129), jnp.float32),
            pltpu.VMEM((_NBUF_R, 32, 128), jnp.float32),
            pltpu.SemaphoreType.DMA((_NBUF_R,)),
            pltpu.SemaphoreType.DMA((_NBUF_R,)),
        ],
        compiler_params=pltpu.CompilerParams(
            use_tc_tiling_on_sc=True, needs_layout_passes=False
        ),
    )
    return kern(tt, tail4)


def _sc_gather(x3, table4):
    kern = pl.kernel(
        _gather_body,
        out_type=jax.ShapeDtypeStruct((HIST, 4, _NW, 8, _CH), jnp.float32),
        mesh=plsc.VectorSubcoreMesh(core_axis_name="c", subcore_axis_name="s"),
        scratch_types=[
            pltpu.VMEM((_NCH, _CH), jnp.int32),
            pltpu.VMEM((_NCH, _CH), jnp.int32),
            pltpu.VMEM((_NBUF, _CH, 128), jnp.float32),
            pltpu.VMEM((_NBUF, 4, 8, _CH), jnp.float32),
            pltpu.SemaphoreType.DMA((_NBUF,)),
            pltpu.SemaphoreType.DMA((_NBUF,)),
        ],
        compiler_params=pltpu.CompilerParams(
            use_tc_tiling_on_sc=False, needs_layout_passes=False
        ),
    )
    return kern(x3, table4)


def kernel(x, embedding_matrix):
    x3 = x.astype(jnp.int32).T.reshape(HIST, _NW, _CH)
    tail4 = embedding_matrix[_VT_FULL * 128:].reshape(16, 128)
    table4 = _sc_relayout(embedding_matrix.T, tail4)
    out5 = _sc_gather(x3, table4)
    return out5.transpose(2, 4, 0, 1, 3).reshape(BATCH, HIST, EMBED_DIM)


# speedup vs baseline: 1.1656x; 1.1410x over previous
import jax
import jax.numpy as jnp
from jax import lax
from jax.experimental import pallas as pl
from jax.experimental.pallas import tpu as pltpu
from jax.experimental.pallas import tpu_sc as plsc

VOCAB = 1000000
EMBED_DIM = 32
BATCH = 4096
HIST = 50

_NC = 2
_NS = 16
_NW = _NC * _NS

_CH = 128
_NCH = HIST
_NBUF = 2


def _gather_body(x_hbm, table_hbm, out_hbm, idx_v, rows_v, t_v, g_sems, o_sems):
    wid = lax.axis_index("s") * _NC + lax.axis_index("c")

    pltpu.sync_copy(x_hbm.at[:, wid], idx_v)

    iotas = [
        lax.broadcasted_iota(jnp.int32, (16,), 0) + g * 16 for g in range(8)
    ]
    cols = [jnp.full((16,), d, jnp.int32) for d in range(EMBED_DIM)]

    def start_gather(j, b):
        pltpu.async_copy(table_hbm.at[idx_v.at[j]], rows_v.at[b], g_sems.at[b])

    def wait_gather(j, b):
        pltpu.make_async_copy(
            table_hbm.at[idx_v.at[j]], rows_v.at[b], g_sems.at[b]
        ).wait()

    def transpose(b):
        for g in range(8):
            vecs = [
                plsc.load_gather(rows_v.at[b], [iotas[g], cols[d]])
                for d in range(EMBED_DIM)
            ]
            for d in range(EMBED_DIM):
                t_v[b, d // 8, d % 8, pl.ds(g * 16, 16)] = vecs[d]

    def start_out(j, b):
        pltpu.async_copy(t_v.at[b], out_hbm.at[j, :, wid], o_sems.at[b])

    def wait_out(j, b):
        pltpu.make_async_copy(
            t_v.at[b], out_hbm.at[j, :, wid], o_sems.at[b]
        ).wait()

    for b in range(_NBUF):
        start_gather(b, b)

    def step(g, _):
        for b in range(_NBUF):
            j = g * _NBUF + b
            wait_gather(j, b)

            @pl.when(j >= _NBUF)
            def _():
                wait_out(j - _NBUF, b)

            transpose(b)

            @pl.when(j + _NBUF < _NCH)
            def _():
                start_gather(j + _NBUF, b)

            start_out(j, b)
        return _

    lax.fori_loop(0, _NCH // _NBUF, step, None)

    for b in range(_NBUF):
        wait_out(_NCH - _NBUF + b, b)


def _sc_gather(x3, table):
    kern = pl.kernel(
        _gather_body,
        out_type=jax.ShapeDtypeStruct((HIST, 4, _NW, 8, _CH), jnp.float32),
        mesh=plsc.VectorSubcoreMesh(core_axis_name="c", subcore_axis_name="s"),
        scratch_types=[
            pltpu.VMEM((_NCH, _CH), jnp.int32),
            pltpu.VMEM((_NBUF, _CH, EMBED_DIM), jnp.float32),
            pltpu.VMEM((_NBUF, 4, 8, _CH), jnp.float32),
            pltpu.SemaphoreType.DMA((_NBUF,)),
            pltpu.SemaphoreType.DMA((_NBUF,)),
        ],
        compiler_params=pltpu.CompilerParams(
            use_tc_tiling_on_sc=False, needs_layout_passes=False
        ),
    )
    return kern(x3, table)


def kernel(x, embedding_matrix):
    x3 = x.astype(jnp.int32).T.reshape(HIST, _NW, _CH)
    out5 = _sc_gather(x3, embedding_matrix)
    return out5.transpose(2, 4, 0, 1, 3).reshape(BATCH, HIST, EMBED_DIM)
